# baseline (speedup 1.0000x reference)
import functools
import math

import jax
import jax.numpy as jnp
from jax import lax
from jax.experimental import pallas as pl
from jax.experimental.pallas import tpu as pltpu
from jax.experimental.pallas import tpu_sc as plsc

B = 128
N = 8192
D = 64
K_SEL = 256
CAND = 512
CANDP = CAND + 16
MIN_I32 = -(2 ** 31)

_f32 = jnp.float32
_i32 = jnp.int32
_u32 = jnp.uint32

_gumbel_cache = None


def _gumbel():
    global _gumbel_cache
    if _gumbel_cache is None:
        _gumbel_cache = jax.random.gumbel(jax.random.key(42), (B, N), dtype=_f32)
    return _gumbel_cache


def _q_body(uav_ref, wq_ref, q_ref):
    q_ref[...] = lax.dot_general(uav_ref[...], wq_ref[...],
                                 (((1,), (0,)), ((), ())),
                                 preferred_element_type=_f32)


def _project_q(uav2d, wq):
    return pl.pallas_call(
        _q_body,
        out_shape=jax.ShapeDtypeStruct((B, D), _f32),
    )(uav2d, wq)


def _scores_body(e_ref, q_ref, wk_ref, out_ref):
    e = e_ref[0]
    qb = q_ref[0, 0]
    k = lax.dot_general(e, wk_ref[...], (((1,), (0,)), ((), ())),
                        preferred_element_type=_f32)
    s = lax.dot_general(k, qb.reshape(D, 1), (((1,), (0,)), ((), ())),
                        preferred_element_type=_f32)
    out_ref[0] = (1.0 / math.sqrt(D)) * jnp.transpose(s, (1, 0))


def _scores(embed_task, q3d, wk):
    return pl.pallas_call(
        _scores_body,
        grid=(B,),
        in_specs=[
            pl.BlockSpec((1, N, D), lambda b: (b, 0, 0)),
            pl.BlockSpec((1, 1, D), lambda b: (b, 0, 0)),
            pl.BlockSpec((D, D), lambda b: (0, 0)),
        ],
        out_specs=pl.BlockSpec((1, 1, N), lambda b: (b, 0, 0)),
        out_shape=jax.ShapeDtypeStruct((B, 1, N), _f32),
    )(embed_task, q3d, wk)


_RB = 8


def _softmax_body(s_ref, g_ref, attn_ref, key_ref, thr_ref, dest_ref):
    s = s_ref[...]
    m = jnp.max(s, axis=1, keepdims=True)
    e = jnp.exp(s - m)
    denom = jnp.sum(e, axis=1, keepdims=True)
    attn = e / denom
    attn_ref[...] = attn
    pert = jnp.log(attn + 1e-20) + g_ref[...]

    bits = lax.bitcast_convert_type(pert, _i32)
    skey = jnp.where(bits >= 0, bits, bits ^ jnp.int32(0x7FFFFFFF))
    key_ref[...] = skey
    ukey = lax.bitcast_convert_type(skey, _u32) ^ jnp.uint32(0x80000000)

    lo0 = jnp.zeros((_RB, 1), _u32)
    hi0 = jnp.full((_RB, 1), 0xFFFFFFFF, _u32)

    def it(_, lohi):
        lo, hi = lohi
        mid = (lo >> 1) + (hi >> 1) + ((lo | hi) & jnp.uint32(1))
        cnt = jnp.sum((ukey >= mid).astype(_i32), axis=1, keepdims=True)
        ge = cnt >= K_SEL
        lo = jnp.where(ge, mid, lo)
        hi = jnp.where(ge, hi, mid - jnp.uint32(1))
        return lo, hi

    lo, _ = lax.fori_loop(0, 32, it, (lo0, hi0))
    thr = lax.bitcast_convert_type(lo ^ jnp.uint32(0x80000000), _i32)
    thr_ref[...] = jnp.broadcast_to(thr, (_RB, 16))

    mi = (ukey >= lo).astype(_i32)
    incl = mi
    sh = 1
    while sh < N:
        incl = incl + jnp.concatenate(
            [jnp.zeros((_RB, sh), _i32), incl[:, :N - sh]], axis=1)
        sh *= 2
    cnt = jnp.minimum(incl[:, N - 1:N], CAND)
    pos = lax.broadcasted_iota(_i32, (_RB, N), 1)
    nc_dest = jnp.minimum(cnt + (pos - incl), CANDP - 1)
    dest = jnp.where(mi > 0,
                     jnp.where(incl <= CAND, incl - 1, CANDP - 1),
                     nc_dest)
    rowid = pl.program_id(0) * _RB + lax.broadcasted_iota(_i32, (_RB, 1), 0)
    dest_ref[...] = dest + rowid * CANDP


def _softmax_perturb(scores2d, gumbel):
    return pl.pallas_call(
        _softmax_body,
        grid=(B // _RB,),
        in_specs=[
            pl.BlockSpec((_RB, N), lambda i: (i, 0)),
            pl.BlockSpec((_RB, N), lambda i: (i, 0)),
        ],
        out_specs=[
            pl.BlockSpec((_RB, N), lambda i: (i, 0)),
            pl.BlockSpec((_RB, N), lambda i: (i, 0)),
            pl.BlockSpec((_RB, 16), lambda i: (i, 0)),
            pl.BlockSpec((_RB, N), lambda i: (i, 0)),
        ],
        out_shape=[
            jax.ShapeDtypeStruct((B, N), _f32),
            jax.ShapeDtypeStruct((B, N), _i32),
            jax.ShapeDtypeStruct((B, 16), _i32),
            jax.ShapeDtypeStruct((B, N), _i32),
        ],
    )(scores2d, gumbel)


_NW = 32
_RPW = B // _NW


def _compact_sc(keys, dest):
    mesh = plsc.VectorSubcoreMesh(core_axis_name="c", subcore_axis_name="s")

    @functools.partial(
        pl.kernel,
        mesh=mesh,
        out_type=[
            jax.ShapeDtypeStruct((B * CANDP,), _i32),
            jax.ShapeDtypeStruct((B * CANDP,), _i32),
        ],
        scratch_types=[
            pltpu.VMEM((N // 128, 128), _i32),
            pltpu.VMEM((N // 128, 128), _i32),
            pltpu.VMEM((N // 128, 128), _i32),
            pltpu.SemaphoreType.DMA,
        ],
    )
    def body(keys_hbm, dest_hbm, iota_hbm, cval_hbm, cidx_hbm,
             row_v, dst_v, io_v, sem):
        wid = lax.axis_index("s") * 2 + lax.axis_index("c")
        pltpu.sync_copy(iota_hbm, io_v)

        for rr in range(_RPW):
            row = wid * _RPW + rr
            pltpu.sync_copy(keys_hbm.at[row], row_v)
            pltpu.sync_copy(dest_hbm.at[row], dst_v)

            def group(g, carry):
                for jj in range(8):
                    j = g * 8 + jj
                    pltpu.async_copy(row_v.at[j], cval_hbm.at[dst_v.at[j]], sem)
                    pltpu.async_copy(io_v.at[j], cidx_hbm.at[dst_v.at[j]], sem)
                for _ in range(2):
                    pltpu.make_async_copy(
                        keys_hbm.at[row].at[pl.ds(0, 8)],
                        row_v.at[pl.ds(0, 8)], sem).wait()
                return carry

            lax.fori_loop(0, N // 128 // 8, group, jnp.int32(0))

    return body(keys.reshape(B, N // 128, 128),
                dest.reshape(B, N // 128, 128),
                jnp.arange(N, dtype=_i32).reshape(N // 128, 128))


def _sort_body(cv_ref, ci_ref, out_ref):
    v = cv_ref[...]
    ix = ci_ref[...]
    pos = lax.broadcasted_iota(_i32, (1, CAND), 1)

    k = 2
    while k <= CAND:
        j = k // 2
        while j >= 1:
            bitj = (pos & j) != 0
            pv = jnp.where(bitj, jnp.roll(v, j, axis=1), jnp.roll(v, -j, axis=1))
            pix = jnp.where(bitj, jnp.roll(ix, j, axis=1), jnp.roll(ix, -j, axis=1))
            gt = (v > pv) | ((v == pv) & (ix < pix))
            descblk = (pos & k) == 0
            lower = ~bitj
            take_self = gt ^ lower ^ descblk
            v = jnp.where(take_self, v, pv)
            ix = jnp.where(take_self, ix, pix)
            j //= 2
        k *= 2
    out_ref[...] = ix[:, :K_SEL]


def _sort_select(cval, cidx):
    return pl.pallas_call(
        _sort_body,
        out_shape=jax.ShapeDtypeStruct((B, K_SEL), _i32),
    )(cval, cidx)


def kernel(embed_task, embed_uav, W_query, W_key):
    uav2d = embed_uav.reshape(B, D)
    q = _project_q(uav2d, W_query)
    q3d = q.reshape(B, 1, D)
    scores = _scores(embed_task, q3d, W_key)
    scores2d = scores.reshape(B, N)
    attn, keys, thr, dest = _softmax_perturb(scores2d, _gumbel())
    cvalf, cidxf = _compact_sc(keys, dest)
    cval = cvalf.reshape(B, CANDP)[:, :CAND]
    cidx = cidxf.reshape(B, CANDP)[:, :CAND]
    selected = _sort_select(cval, cidx)
    return attn.reshape(B, N, 1), selected


# speedup vs baseline: 14.2934x; 14.2934x over previous
import functools
import math

import jax
import jax.numpy as jnp
from jax import lax
from jax.experimental import pallas as pl
from jax.experimental.pallas import tpu as pltpu
from jax.experimental.pallas import tpu_sc as plsc

B = 128
N = 8192
D = 64
K_SEL = 256
CAND = 512
CANDP = CAND + 16
MIN_I32 = -(2 ** 31)

_f32 = jnp.float32
_i32 = jnp.int32
_u32 = jnp.uint32

_gumbel_cache = None


def _gumbel():
    global _gumbel_cache
    if _gumbel_cache is None:
        _gumbel_cache = jax.random.gumbel(jax.random.key(42), (B, N), dtype=_f32)
    return _gumbel_cache


def _q_body(uav_ref, wq_ref, q_ref):
    q_ref[...] = lax.dot_general(uav_ref[...], wq_ref[...],
                                 (((1,), (0,)), ((), ())),
                                 preferred_element_type=_f32)


def _project_q(uav2d, wq):
    return pl.pallas_call(
        _q_body,
        out_shape=jax.ShapeDtypeStruct((B, D), _f32),
    )(uav2d, wq)


def _scores_body(e_ref, q_ref, wk_ref, out_ref):
    e = e_ref[0]
    qb = q_ref[0, 0]
    k = lax.dot_general(e, wk_ref[...], (((1,), (0,)), ((), ())),
                        preferred_element_type=_f32)
    s = lax.dot_general(k, qb.reshape(D, 1), (((1,), (0,)), ((), ())),
                        preferred_element_type=_f32)
    out_ref[0] = (1.0 / math.sqrt(D)) * jnp.transpose(s, (1, 0))


def _scores(embed_task, q3d, wk):
    return pl.pallas_call(
        _scores_body,
        grid=(B,),
        in_specs=[
            pl.BlockSpec((1, N, D), lambda b: (b, 0, 0)),
            pl.BlockSpec((1, 1, D), lambda b: (b, 0, 0)),
            pl.BlockSpec((D, D), lambda b: (0, 0)),
        ],
        out_specs=pl.BlockSpec((1, 1, N), lambda b: (b, 0, 0)),
        out_shape=jax.ShapeDtypeStruct((B, 1, N), _f32),
    )(embed_task, q3d, wk)


_RB = 8


def _softmax_body(s_ref, g_ref, attn_ref, key_ref, thr_ref, dest_ref):
    s = s_ref[...]
    m = jnp.max(s, axis=1, keepdims=True)
    e = jnp.exp(s - m)
    denom = jnp.sum(e, axis=1, keepdims=True)
    attn = e / denom
    attn_ref[...] = attn
    pert = jnp.log(attn + 1e-20) + g_ref[...]

    bits = lax.bitcast_convert_type(pert, _i32)
    skey = jnp.where(bits >= 0, bits, bits ^ jnp.int32(0x7FFFFFFF))
    key_ref[...] = skey
    ukey = lax.bitcast_convert_type(skey, _u32) ^ jnp.uint32(0x80000000)

    lo0 = jnp.zeros((_RB, 1), _u32)
    hi0 = jnp.full((_RB, 1), 0xFFFFFFFF, _u32)

    def it(_, lohi):
        lo, hi = lohi
        mid = (lo >> 1) + (hi >> 1) + ((lo | hi) & jnp.uint32(1))
        cnt = jnp.sum((ukey >= mid).astype(_i32), axis=1, keepdims=True)
        ge = cnt >= K_SEL
        lo = jnp.where(ge, mid, lo)
        hi = jnp.where(ge, hi, mid - jnp.uint32(1))
        return lo, hi

    lo, _ = lax.fori_loop(0, 32, it, (lo0, hi0))
    thr = lax.bitcast_convert_type(lo ^ jnp.uint32(0x80000000), _i32)
    thr_ref[...] = jnp.broadcast_to(thr, (_RB, 16))

    mi = (ukey >= lo).astype(_i32)
    incl = mi
    sh = 1
    while sh < N:
        incl = incl + jnp.concatenate(
            [jnp.zeros((_RB, sh), _i32), incl[:, :N - sh]], axis=1)
        sh *= 2
    cnt = jnp.minimum(incl[:, N - 1:N], CAND)
    pos = lax.broadcasted_iota(_i32, (_RB, N), 1)
    nc_dest = jnp.minimum(cnt + (pos - incl), CANDP - 1)
    dest = jnp.where(mi > 0,
                     jnp.where(incl <= CAND, incl - 1, CANDP - 1),
                     nc_dest)
    rowid = pl.program_id(0) * _RB + lax.broadcasted_iota(_i32, (_RB, 1), 0)
    dest_ref[...] = dest + (rowid // 8) * CANDP


def _softmax_perturb(scores2d, gumbel):
    return pl.pallas_call(
        _softmax_body,
        grid=(B // _RB,),
        in_specs=[
            pl.BlockSpec((_RB, N), lambda i: (i, 0)),
            pl.BlockSpec((_RB, N), lambda i: (i, 0)),
        ],
        out_specs=[
            pl.BlockSpec((_RB, N), lambda i: (i, 0)),
            pl.BlockSpec((_RB, N), lambda i: (i, 0)),
            pl.BlockSpec((_RB, 16), lambda i: (i, 0)),
            pl.BlockSpec((_RB, N), lambda i: (i, 0)),
        ],
        out_shape=[
            jax.ShapeDtypeStruct((B, N), _f32),
            jax.ShapeDtypeStruct((B, N), _i32),
            jax.ShapeDtypeStruct((B, 16), _i32),
            jax.ShapeDtypeStruct((B, N), _i32),
        ],
    )(scores2d, gumbel)


_NW = 32
_RPW = B // _NW


def _compact_sc(keys, dest):
    mesh = plsc.VectorSubcoreMesh(core_axis_name="c", subcore_axis_name="s")

    @functools.partial(
        pl.kernel,
        mesh=mesh,
        out_type=[
            jax.ShapeDtypeStruct((B * CANDP,), _i32),
            jax.ShapeDtypeStruct((B * CANDP,), _i32),
        ],
        scratch_types=[
            pltpu.VMEM((N // 128, 128), _i32),
            pltpu.VMEM((N // 128, 128), _i32),
            pltpu.VMEM((N // 128, 128), _i32),
            pltpu.VMEM_SHARED((16 * CANDP,), _i32),
            pltpu.VMEM_SHARED((16 * CANDP,), _i32),
            pltpu.VMEM((CANDP,), _i32),
            pltpu.VMEM((CANDP,), _i32),
            pltpu.SemaphoreType.DMA,
        ],
    )
    def body(keys_hbm, dest_hbm, iota_hbm, cval_hbm, cidx_hbm,
             row_v, dst_v, io_v, shv, shi, bv, bi, sem):
        wid = lax.axis_index("s") * 2 + lax.axis_index("c")
        sidx = lax.axis_index("s")
        pltpu.sync_copy(iota_hbm, io_v)

        for rr in range(_RPW):
            row = wid * _RPW + rr
            pltpu.sync_copy(keys_hbm.at[row], row_v)
            pltpu.sync_copy(dest_hbm.at[row], dst_v)

            def group(g, carry):
                for jj in range(8):
                    j = g * 8 + jj
                    pltpu.async_copy(row_v.at[j], shv.at[dst_v.at[j]], sem)
                    pltpu.async_copy(io_v.at[j], shi.at[dst_v.at[j]], sem)
                for _ in range(2):
                    pltpu.make_async_copy(
                        keys_hbm.at[row].at[pl.ds(0, 8)],
                        row_v.at[pl.ds(0, 8)], sem).wait()
                return carry

            lax.fori_loop(0, N // 128 // 8, group, jnp.int32(0))
            pltpu.sync_copy(shv.at[pl.ds(sidx * CANDP, CANDP)], bv)
            pltpu.sync_copy(shi.at[pl.ds(sidx * CANDP, CANDP)], bi)
            pltpu.sync_copy(bv, cval_hbm.at[pl.ds(row * CANDP, CANDP)])
            pltpu.sync_copy(bi, cidx_hbm.at[pl.ds(row * CANDP, CANDP)])

    return body(keys.reshape(B, N // 128, 128),
                dest.reshape(B, N // 128, 128),
                jnp.arange(N, dtype=_i32).reshape(N // 128, 128))


def _sort_body(cv_ref, ci_ref, out_ref):
    v = cv_ref[...]
    ix = ci_ref[...]
    pos = lax.broadcasted_iota(_i32, (1, CAND), 1)

    k = 2
    while k <= CAND:
        j = k // 2
        while j >= 1:
            bitj = (pos & j) != 0
            pv = jnp.where(bitj, jnp.roll(v, j, axis=1), jnp.roll(v, -j, axis=1))
            pix = jnp.where(bitj, jnp.roll(ix, j, axis=1), jnp.roll(ix, -j, axis=1))
            gt = (v > pv) | ((v == pv) & (ix < pix))
            descblk = (pos & k) == 0
            lower = ~bitj
            take_self = gt ^ lower ^ descblk
            v = jnp.where(take_self, v, pv)
            ix = jnp.where(take_self, ix, pix)
            j //= 2
        k *= 2
    out_ref[...] = ix[:, :K_SEL]


def _sort_select(cval, cidx):
    return pl.pallas_call(
        _sort_body,
        out_shape=jax.ShapeDtypeStruct((B, K_SEL), _i32),
    )(cval, cidx)


def kernel(embed_task, embed_uav, W_query, W_key):
    uav2d = embed_uav.reshape(B, D)
    q = _project_q(uav2d, W_query)
    q3d = q.reshape(B, 1, D)
    scores = _scores(embed_task, q3d, W_key)
    scores2d = scores.reshape(B, N)
    attn, keys, thr, dest = _softmax_perturb(scores2d, _gumbel())
    cvalf, cidxf = _compact_sc(keys, dest)
    cval = cvalf.reshape(B, CANDP)[:, :CAND]
    cidx = cidxf.reshape(B, CANDP)[:, :CAND]
    selected = _sort_select(cval, cidx)
    return attn.reshape(B, N, 1), selected


# speedup vs baseline: 14.3346x; 1.0029x over previous
import functools
import math

import jax
import jax.numpy as jnp
from jax import lax
from jax.experimental import pallas as pl
from jax.experimental.pallas import tpu as pltpu
from jax.experimental.pallas import tpu_sc as plsc

B = 128
N = 8192
D = 64
K_SEL = 256
CAND = 512
CANDP = CAND + 16
_f32 = jnp.float32
_i32 = jnp.int32
_u32 = jnp.uint32

_gumbel_cache = None


def _gumbel():
    global _gumbel_cache
    if _gumbel_cache is None:
        _gumbel_cache = jax.random.gumbel(jax.random.key(42), (B, N), dtype=_f32)
    return _gumbel_cache


def _q_body(uav_ref, wq_ref, q_ref):
    q_ref[...] = lax.dot_general(uav_ref[...], wq_ref[...],
                                 (((1,), (0,)), ((), ())),
                                 preferred_element_type=_f32)


def _project_q(uav2d, wq):
    return pl.pallas_call(
        _q_body,
        out_shape=jax.ShapeDtypeStruct((B, D), _f32),
    )(uav2d, wq)


def _scores_body(e_ref, q_ref, wk_ref, out_ref):
    e = e_ref[0]
    qb = q_ref[0, 0]
    k = lax.dot_general(e, wk_ref[...], (((1,), (0,)), ((), ())),
                        preferred_element_type=_f32)
    s = lax.dot_general(k, qb.reshape(D, 1), (((1,), (0,)), ((), ())),
                        preferred_element_type=_f32)
    out_ref[0] = (1.0 / math.sqrt(D)) * jnp.transpose(s, (1, 0))


def _scores(embed_task, q3d, wk):
    return pl.pallas_call(
        _scores_body,
        grid=(B,),
        in_specs=[
            pl.BlockSpec((1, N, D), lambda b: (b, 0, 0)),
            pl.BlockSpec((1, 1, D), lambda b: (b, 0, 0)),
            pl.BlockSpec((D, D), lambda b: (0, 0)),
        ],
        out_specs=pl.BlockSpec((1, 1, N), lambda b: (b, 0, 0)),
        out_shape=jax.ShapeDtypeStruct((B, 1, N), _f32),
    )(embed_task, q3d, wk)


_RB = 8


def _softmax_body(s_ref, g_ref, attn_ref, key_ref, dest_ref):
    s = s_ref[...]
    m = jnp.max(s, axis=1, keepdims=True)
    e = jnp.exp(s - m)
    denom = jnp.sum(e, axis=1, keepdims=True)
    attn = e / denom
    attn_ref[...] = attn
    pert = jnp.log(attn + 1e-20) + g_ref[...]

    bits = lax.bitcast_convert_type(pert, _i32)
    skey = jnp.where(bits >= 0, bits, bits ^ jnp.int32(0x7FFFFFFF))
    key_ref[...] = skey
    ukey = lax.bitcast_convert_type(skey, _u32) ^ jnp.uint32(0x80000000)

    lo0 = jnp.zeros((_RB, 1), _u32)
    hi0 = jnp.full((_RB, 1), 0xFFFFFFFF, _u32)

    def it(_, lohi):
        lo, hi = lohi
        mid = (lo >> 1) + (hi >> 1) + ((lo | hi) & jnp.uint32(1))
        cnt = jnp.sum((ukey >= mid).astype(_i32), axis=1, keepdims=True)
        ge = cnt >= K_SEL
        lo = jnp.where(ge, mid, lo)
        hi = jnp.where(ge, hi, mid - jnp.uint32(1))
        return lo, hi

    lo, _ = lax.fori_loop(0, 32, it, (lo0, hi0))

    mi = (ukey >= lo).astype(_i32)
    incl = mi
    sh = 1
    while sh < N:
        incl = incl + jnp.concatenate(
            [jnp.zeros((_RB, sh), _i32), incl[:, :N - sh]], axis=1)
        sh *= 2
    cnt = jnp.minimum(incl[:, N - 1:N], CAND)
    pos = lax.broadcasted_iota(_i32, (_RB, N), 1)
    nc_dest = jnp.minimum(cnt + (pos - incl), CANDP - 1)
    dest = jnp.where(mi > 0,
                     jnp.where(incl <= CAND, incl - 1, CANDP - 1),
                     nc_dest)
    rowid = pl.program_id(0) * _RB + lax.broadcasted_iota(_i32, (_RB, 1), 0)
    dest_ref[...] = dest + (rowid // 8) * CANDP


def _softmax_perturb(scores2d, gumbel):
    return pl.pallas_call(
        _softmax_body,
        grid=(B // _RB,),
        in_specs=[
            pl.BlockSpec((_RB, N), lambda i: (i, 0)),
            pl.BlockSpec((_RB, N), lambda i: (i, 0)),
        ],
        out_specs=[
            pl.BlockSpec((_RB, N), lambda i: (i, 0)),
            pl.BlockSpec((_RB, N), lambda i: (i, 0)),
            pl.BlockSpec((_RB, N), lambda i: (i, 0)),
        ],
        out_shape=[
            jax.ShapeDtypeStruct((B, N), _f32),
            jax.ShapeDtypeStruct((B, N), _i32),
            jax.ShapeDtypeStruct((B, N), _i32),
        ],
    )(scores2d, gumbel)


_NW = 32
_RPW = B // _NW


def _compact_sc(keys, dest):
    mesh = plsc.VectorSubcoreMesh(core_axis_name="c", subcore_axis_name="s")

    @functools.partial(
        pl.kernel,
        mesh=mesh,
        out_type=[
            jax.ShapeDtypeStruct((B * CANDP,), _i32),
            jax.ShapeDtypeStruct((B * CANDP,), _i32),
        ],
        scratch_types=[
            pltpu.VMEM((N // 128, 128), _i32),
            pltpu.VMEM((N // 128, 128), _i32),
            pltpu.VMEM((N // 128, 128), _i32),
            pltpu.VMEM_SHARED((16 * CANDP,), _i32),
            pltpu.VMEM_SHARED((16 * CANDP,), _i32),
            pltpu.VMEM((CANDP,), _i32),
            pltpu.VMEM((CANDP,), _i32),
            pltpu.SemaphoreType.DMA,
        ],
    )
    def body(keys_hbm, dest_hbm, iota_hbm, cval_hbm, cidx_hbm,
             row_v, dst_v, io_v, shv, shi, bv, bi, sem):
        wid = lax.axis_index("s") * 2 + lax.axis_index("c")
        sidx = lax.axis_index("s")
        pltpu.sync_copy(iota_hbm, io_v)

        for rr in range(_RPW):
            row = wid * _RPW + rr
            pltpu.sync_copy(keys_hbm.at[row], row_v)
            pltpu.sync_copy(dest_hbm.at[row], dst_v)

            def group(g, carry):
                for jj in range(8):
                    j = g * 8 + jj
                    pltpu.async_copy(row_v.at[j], shv.at[dst_v.at[j]], sem)
                    pltpu.async_copy(io_v.at[j], shi.at[dst_v.at[j]], sem)
                for _ in range(2):
                    pltpu.make_async_copy(
                        keys_hbm.at[row].at[pl.ds(0, 8)],
                        row_v.at[pl.ds(0, 8)], sem).wait()
                return carry

            lax.fori_loop(0, N // 128 // 8, group, jnp.int32(0))
            pltpu.sync_copy(shv.at[pl.ds(sidx * CANDP, CANDP)], bv)
            pltpu.sync_copy(shi.at[pl.ds(sidx * CANDP, CANDP)], bi)
            pltpu.sync_copy(bv, cval_hbm.at[pl.ds(row * CANDP, CANDP)])
            pltpu.sync_copy(bi, cidx_hbm.at[pl.ds(row * CANDP, CANDP)])

    return body(keys.reshape(B, N // 128, 128),
                dest.reshape(B, N // 128, 128),
                jnp.arange(N, dtype=_i32).reshape(N // 128, 128))


def _sort_body(cv_ref, ci_ref, out_ref):
    v = cv_ref[...]
    ix = ci_ref[...]
    pos = lax.broadcasted_iota(_i32, (1, CAND), 1)

    k = 2
    while k <= CAND:
        j = k // 2
        while j >= 1:
            bitj = (pos & j) != 0
            pv = jnp.where(bitj, jnp.roll(v, j, axis=1), jnp.roll(v, -j, axis=1))
            pix = jnp.where(bitj, jnp.roll(ix, j, axis=1), jnp.roll(ix, -j, axis=1))
            gt = (v > pv) | ((v == pv) & (ix < pix))
            descblk = (pos & k) == 0
            lower = ~bitj
            take_self = gt ^ lower ^ descblk
            v = jnp.where(take_self, v, pv)
            ix = jnp.where(take_self, ix, pix)
            j //= 2
        k *= 2
    out_ref[...] = ix[:, :K_SEL]


def _sort_select(cval, cidx):
    return pl.pallas_call(
        _sort_body,
        out_shape=jax.ShapeDtypeStruct((B, K_SEL), _i32),
    )(cval, cidx)


def kernel(embed_task, embed_uav, W_query, W_key):
    uav2d = embed_uav.reshape(B, D)
    q = _project_q(uav2d, W_query)
    q3d = q.reshape(B, 1, D)
    scores = _scores(embed_task, q3d, W_key)
    scores2d = scores.reshape(B, N)
    attn, keys, dest = _softmax_perturb(scores2d, _gumbel())
    cvalf, cidxf = _compact_sc(keys, dest)
    cval = cvalf.reshape(B, CANDP)[:, :CAND]
    cidx = cidxf.reshape(B, CANDP)[:, :CAND]
    selected = _sort_select(cval, cidx)
    return attn.reshape(B, N, 1), selected
